# baseline (speedup 1.0000x reference)
import functools

import jax
import jax.numpy as jnp
from jax import lax
from jax.experimental import pallas as pl
from jax.experimental.pallas import tpu as pltpu
from jax.experimental.pallas import tpu_sc as plsc

NUM_CORES = 2
NUM_SUBCORES = 16
NUM_WORKERS = NUM_CORES * NUM_SUBCORES

GATHER = 128
IPW = 512
NJ = 200
KT, KR = 4, 8
IT = IPW // GATHER


def _sc_embedding_lookup(table, idx3):
    V, D = table.shape
    assert D == KT * KR and idx3.shape == (NJ, 128, 128)
    npairs = NJ // 2

    mesh = plsc.VectorSubcoreMesh(core_axis_name="c", subcore_axis_name="s")

    @functools.partial(
        pl.kernel,
        out_type=jax.ShapeDtypeStruct((NJ, KT, 128, KR, GATHER), jnp.float32),
        mesh=mesh,
        compiler_params=pltpu.CompilerParams(
            use_tc_tiling_on_sc=False, needs_layout_passes=False
        ),
        scratch_types=[
            pltpu.VMEM((2, IT, GATHER), jnp.int32),
            pltpu.VMEM((2, IPW, D), jnp.float32),
            pltpu.VMEM((2, KT, IT, KR, GATHER), jnp.float32),
            pltpu.SemaphoreType.DMA,
            pltpu.SemaphoreType.DMA,
            pltpu.SemaphoreType.DMA,
            pltpu.SemaphoreType.DMA,
        ],
    )
    def k(table_hbm, idx_hbm, out_hbm, idx_v, rows_v, trans_v,
          gsem0, gsem1, ssem0, ssem1):
        wid = lax.axis_index("s") * NUM_CORES + lax.axis_index("c")
        it0 = pl.multiple_of(wid * IT, IT)
        lanes = jnp.arange(16, dtype=jnp.int32)

        def idx_load(j, slot):
            pltpu.sync_copy(idx_hbm.at[j, pl.ds(it0, IT)], idx_v.at[slot])

        def fire_gathers(slot, sem):
            for r in range(IT):
                pltpu.async_copy(
                    table_hbm.at[idx_v.at[slot, r]],
                    rows_v.at[slot, pl.ds(r * GATHER, GATHER)],
                    sem,
                )

        def drain_gathers(slot, sem):
            pltpu.make_async_copy(
                table_hbm.at[pl.ds(0, IPW)], rows_v.at[slot], sem
            ).wait()

        def transpose(slot):
            rows = rows_v.at[slot]
            for itl in range(IT):

                def body(ib8, carry):
                    row_ids = itl * GATHER + ib8 * 16 + lanes
                    for kt in range(KT):
                        for kr in range(KR):
                            col = kt * KR + kr
                            x = plsc.load_gather(
                                rows, [row_ids, jnp.full((16,), col, jnp.int32)]
                            )
                            trans_v[slot, kt, itl, kr, pl.ds(ib8 * 16, 16)] = x
                    return carry

                lax.fori_loop(0, GATHER // 16, body, 0)

        def fire_store(j, slot, sem):
            pltpu.async_copy(
                trans_v.at[slot],
                out_hbm.at[j, :, pl.ds(it0, IT)],
                sem,
            )

        def drain_store(slot, sem):
            pltpu.make_async_copy(
                trans_v.at[slot], out_hbm.at[0, :, pl.ds(0, IT)], sem
            ).wait()

        idx_load(0, 0)
        fire_gathers(0, gsem0)

        def pair(jp, carry):
            j0 = 2 * jp
            idx_load(j0 + 1, 1)
            drain_gathers(0, gsem0)
            fire_gathers(1, gsem1)

            @pl.when(jp >= 1)
            def _():
                drain_store(0, ssem0)

            transpose(0)
            fire_store(j0, 0, ssem0)

            drain_gathers(1, gsem1)

            @pl.when(jp < npairs - 1)
            def _():
                idx_load(j0 + 2, 0)
                fire_gathers(0, gsem0)

            @pl.when(jp >= 1)
            def _():
                drain_store(1, ssem1)

            transpose(1)
            fire_store(j0 + 1, 1, ssem1)
            return carry

        lax.fori_loop(0, npairs, pair, 0)
        drain_store(0, ssem0)
        drain_store(1, ssem1)

    return k(table, idx3)


def kernel(indices, weight):
    ni, nj = indices.shape
    idx3 = jnp.transpose(indices).reshape(nj, ni // 128, 128).astype(jnp.int32)
    o5 = _sc_embedding_lookup(weight, idx3)
    out = jnp.transpose(o5, (2, 4, 0, 1, 3)).reshape(ni, nj, KT * KR)
    return out


# speedup vs baseline: 1.2652x; 1.2652x over previous
import functools

import jax
import jax.numpy as jnp
from jax import lax
from jax.experimental import pallas as pl
from jax.experimental.pallas import tpu as pltpu
from jax.experimental.pallas import tpu_sc as plsc

NUM_CORES = 2
NUM_SUBCORES = 16
NUM_WORKERS = NUM_CORES * NUM_SUBCORES

CHUNK = 512
OCT = 8


def _sc_gather(table, idx_flat):
    B = idx_flat.shape[0]
    D = table.shape[1]
    rows_per_w = B // NUM_WORKERS
    chunks_per_w = rows_per_w // CHUNK
    nquads = chunks_per_w // 4
    nocts = chunks_per_w // OCT
    assert rows_per_w % (CHUNK * OCT) == 0

    mesh = plsc.VectorSubcoreMesh(core_axis_name="c", subcore_axis_name="s")

    @functools.partial(
        pl.kernel,
        out_type=jax.ShapeDtypeStruct((B, D), jnp.float32),
        mesh=mesh,
        compiler_params=pltpu.CompilerParams(use_tc_tiling_on_sc=False),
        scratch_types=[
            pltpu.VMEM((2, OCT * CHUNK), jnp.int32),
            pltpu.VMEM((4, CHUNK, D), jnp.float32),
            pltpu.SemaphoreType.DMA,
            pltpu.SemaphoreType.DMA,
            pltpu.SemaphoreType.DMA,
            pltpu.SemaphoreType.DMA,
            pltpu.SemaphoreType.DMA,
            pltpu.SemaphoreType.DMA,
            pltpu.SemaphoreType.DMA,
            pltpu.SemaphoreType.DMA,
            pltpu.SemaphoreType.DMA,
        ],
    )
    def k(table_hbm, idx_hbm, out_hbm, idx_v, rows_v,
          isem, g0, g1, g2, g3, s0, s1, s2, s3):
        gsem = (g0, g1, g2, g3)
        ssem = (s0, s1, s2, s3)
        wid = lax.axis_index("s") * NUM_CORES + lax.axis_index("c")
        row0 = wid * rows_per_w

        def fire_idx_load(o, islot):
            base = pl.multiple_of(row0 + o * OCT * CHUNK, CHUNK)
            pltpu.async_copy(
                idx_hbm.at[pl.ds(base, OCT * CHUNK)], idx_v.at[islot], isem
            )

        def drain_idx_load(islot):
            pltpu.make_async_copy(
                idx_hbm.at[pl.ds(0, OCT * CHUNK)], idx_v.at[islot], isem
            ).wait()

        def fire_gather(islot, orow, b, sem):
            pltpu.async_copy(
                table_hbm.at[idx_v.at[islot, pl.ds(orow * CHUNK, CHUNK)]],
                rows_v.at[b],
                sem,
            )

        def drain_gather(b, sem):
            pltpu.make_async_copy(
                table_hbm.at[pl.ds(0, CHUNK)], rows_v.at[b], sem
            ).wait()

        def fire_store(c, b, sem):
            base = pl.multiple_of(row0 + c * CHUNK, CHUNK)
            pltpu.async_copy(rows_v.at[b], out_hbm.at[pl.ds(base, CHUNK)], sem)

        def drain_store(b, sem):
            pltpu.make_async_copy(
                rows_v.at[b], out_hbm.at[pl.ds(0, CHUNK)], sem
            ).wait()

        fire_idx_load(0, 0)
        drain_idx_load(0)

        def quad(q, carry):
            o = q // 2
            islot = lax.rem(o, 2)
            qh = lax.rem(q, 2)
            even_q = qh == 0
            for b in range(4):
                c = 4 * q + b
                orow = 4 * qh + b

                if b == 0:

                    @pl.when(jnp.logical_and(even_q, q > 0))
                    def _():
                        drain_idx_load(islot)

                @pl.when(q >= 1)
                def _():
                    drain_store(b, ssem[b])

                fire_gather(islot, orow, b, gsem[b])

                if b == 2:

                    @pl.when(jnp.logical_and(even_q, o + 1 < nocts))
                    def _():
                        fire_idx_load(o + 1, 1 - islot)

                bl = (b + 2) % 4
                if b >= 2:
                    drain_gather(bl, gsem[bl])
                    fire_store(c - 2, bl, ssem[bl])
                else:

                    @pl.when(q >= 1)
                    def _():
                        drain_gather(bl, gsem[bl])
                        fire_store(c - 2, bl, ssem[bl])

            return carry

        lax.fori_loop(0, nquads, quad, 0)

        last = chunks_per_w
        for (c, b) in ((last - 2, 2), (last - 1, 3)):
            drain_gather(b, gsem[b])
            fire_store(c, b, ssem[b])
        for b in range(4):
            drain_store(b, ssem[b])

    return k(table, idx_flat)


def kernel(indices, weight):
    B = indices.shape[0] * indices.shape[1]
    idx_flat = indices.reshape(B).astype(jnp.int32)
    out = _sc_gather(weight, idx_flat)
    return out.reshape(indices.shape + (weight.shape[1],))
